# baseline (speedup 1.0000x reference)
import jax
import jax.numpy as jnp
from jax import lax
from jax.experimental import pallas as pl
from jax.experimental.pallas import tpu as pltpu

B, C, H, W = 32, 768, 32, 32
HW = H * W
CHANNEL_IDX = 0
MIN_CR, MAX_CR, ALPHA = 0.3, 1.0, 0.7


def _dotT(a, b):
    return lax.dot_general(a, b, (((1,), (1,)), ((), ())),
                           preferred_element_type=jnp.float32)


def _bf(v):
    return v.astype(jnp.bfloat16).astype(jnp.float32)


def _body(scal_ref, x_ref, cemb_ref, snr_w1r_ref, snr_b1r_ref, snr_w2_ref,
          snr_b2r_ref, sem_w_ref, cond_w_ref, out_w_ref, rc_w1_ref,
          rc_b1r_ref, rc_w2_ref, rc_b2r_ref, rc_w3_ref, eye_ref, lt_ref,
          ones_ref, out_ref, cr_ref):
    f32 = jnp.float32
    base_cr = scal_ref[0, 0]
    snr_db = scal_ref[0, 1]
    rc_b3 = scal_ref[0, 2]
    relu = lambda v: jnp.maximum(v, 0.0)

    xb = x_ref[0]
    sem_row = jnp.sum(xb, axis=0, keepdims=True) * f32(1.0 / HW)

    ch_row = _bf(cemb_ref[CHANNEL_IDX:CHANNEL_IDX + 1, :])
    snr_norm = snr_db / 28.0
    h1 = _bf(relu(snr_w1r_ref[...] * snr_norm + snr_b1r_ref[...]))
    snr_row = _bf(relu(_dotT(h1, snr_w2_ref[...]) + snr_b2r_ref[...]))
    cond_row = jnp.concatenate([ch_row, snr_row], axis=1)

    fused = _bf(relu(_dotT(sem_row, sem_w_ref[...])
                     + _dotT(cond_row, cond_w_ref[...])))
    w_row = 1.0 / (1.0 + jnp.exp(-_dotT(fused, out_w_ref[...])))

    rc_in = jnp.concatenate([_bf(sem_row), cond_row], axis=1)
    r1 = _bf(relu(_dotT(rc_in, rc_w1_ref[...]) + rc_b1r_ref[...]))
    r2 = _bf(relu(_dotT(r1, rc_w2_ref[...]) + rc_b2r_ref[...]))
    raw = 1.0 / (1.0 + jnp.exp(-(_dotT(r2, rc_w3_ref[...])[0, 0] + rc_b3)))
    dyn = MIN_CR + (MAX_CR - MIN_CR) * raw
    cr = jnp.clip((1.0 - ALPHA) * base_cr + ALPHA * dyn, MIN_CR, MAX_CR)
    cr_safe = jnp.clip(cr, 0.001, 1.0)
    kf = jnp.clip(jnp.round(cr_safe * float(C)), 1.0, float(C))

    w_col = lax.dot_general(eye_ref[...], w_row, (((1,), (1,)), ((), ())),
                            precision=lax.Precision.HIGHEST,
                            preferred_element_type=f32)
    beats = ((w_row > w_col).astype(f32)
             + (w_row == w_col).astype(f32) * lt_ref[...])
    rank_row = lax.dot_general(ones_ref[...], beats,
                               (((1,), (1,)), ((), ())),
                               preferred_element_type=f32)
    mask_row = (rank_row < kf).astype(f32)

    out_ref[0] = xb * mask_row
    cr_ref[...] = jnp.full((1, 8, 128), cr, f32)


def _lt_const():
    r = jnp.arange(C)
    return (r[None, :] < r[:, None]).astype(jnp.float32)


def kernel(x, channel_embed, snr_w1, snr_b1, snr_w2, snr_b2, sem_proj_w,
           cond_proj_w, out_proj_w, rc_w1, rc_b1, rc_w2, rc_b2, rc_w3, rc_b3,
           base_cr, snr_db):
    f32 = jnp.float32
    xt = jnp.transpose(x, (0, 2, 3, 1)).reshape(B, HW, C)
    scal = jnp.stack([jnp.asarray(base_cr, f32), jnp.asarray(snr_db, f32),
                      rc_b3.astype(f32)[0]]).reshape(1, 3)
    row = lambda v: v.reshape(1, -1)

    full = lambda s: pl.BlockSpec(s, lambda i: (0,) * len(s))
    out3, cr_buf = pl.pallas_call(
        _body,
        grid=(B,),
        in_specs=[
            pl.BlockSpec(memory_space=pltpu.SMEM),
            pl.BlockSpec((1, HW, C), lambda i: (i, 0, 0)),
            full(channel_embed.shape), full((1, 16)), full((1, 16)),
            full(snr_w2.shape), full((1, 16)),
            full(sem_proj_w.shape), full(cond_proj_w.shape),
            full(out_proj_w.shape),
            full(rc_w1.shape), full((1, rc_b1.shape[0])), full(rc_w2.shape),
            full((1, rc_b2.shape[0])), full(rc_w3.shape),
            full((C, C)), full((C, C)), full((1, C)),
        ],
        out_specs=[
            pl.BlockSpec((1, HW, C), lambda i: (i, 0, 0)),
            pl.BlockSpec((1, 8, 128), lambda i: (i, 0, 0)),
        ],
        out_shape=[
            jax.ShapeDtypeStruct((B, HW, C), f32),
            jax.ShapeDtypeStruct((B, 8, 128), f32),
        ],
        compiler_params=pltpu.CompilerParams(
            dimension_semantics=("arbitrary",)),
    )(scal, xt, channel_embed, row(snr_w1[:, 0]), row(snr_b1), snr_w2,
      row(snr_b2), sem_proj_w, cond_proj_w, out_proj_w,
      rc_w1, row(rc_b1), rc_w2, row(rc_b2), rc_w3,
      jnp.eye(C, dtype=f32), _lt_const(), jnp.ones((1, C), f32))

    out = jnp.transpose(out3.reshape(B, H, W, C), (0, 3, 1, 2))
    return out, cr_buf[:, 0, 0]


# speedup vs baseline: 1.0265x; 1.0265x over previous
import jax
import jax.numpy as jnp
import numpy as np
from jax import lax
from jax.experimental import pallas as pl
from jax.experimental.pallas import tpu as pltpu

B, C, H, W = 32, 768, 32, 32
HW = H * W
CHANNEL_IDX = 0
MIN_CR, MAX_CR, ALPHA = 0.3, 1.0, 0.7


def _dotT(a, b):
    return lax.dot_general(a, b, (((1,), (1,)), ((), ())),
                           preferred_element_type=jnp.float32)


def _bf(v):
    return v.astype(jnp.bfloat16).astype(jnp.float32)


def _body(scal_ref, x_ref, cemb_ref, snr_w1r_ref, snr_b1r_ref, snr_w2_ref,
          snr_b2r_ref, sem_w_ref, cond_w_ref, out_w_ref, rc_w1_ref,
          rc_b1r_ref, rc_w2_ref, rc_b2r_ref, rc_w3_ref, eye_ref, lt_ref,
          ones_ref, out_ref, cr_ref):
    f32 = jnp.float32
    base_cr = scal_ref[0, 0]
    snr_db = scal_ref[0, 1]
    rc_b3 = scal_ref[0, 2]
    relu = lambda v: jnp.maximum(v, 0.0)

    xb = x_ref[0]
    sem_row = jnp.sum(xb, axis=0, keepdims=True) * f32(1.0 / HW)

    ch_row = _bf(cemb_ref[CHANNEL_IDX:CHANNEL_IDX + 1, :])
    snr_norm = snr_db / 28.0
    h1 = _bf(relu(snr_w1r_ref[...] * snr_norm + snr_b1r_ref[...]))
    snr_row = _bf(relu(_dotT(h1, snr_w2_ref[...]) + snr_b2r_ref[...]))
    cond_row = jnp.concatenate([ch_row, snr_row], axis=1)

    fused = _bf(relu(_dotT(sem_row, sem_w_ref[...])
                     + _dotT(cond_row, cond_w_ref[...])))
    w_row = 1.0 / (1.0 + jnp.exp(-_dotT(fused, out_w_ref[...])))

    rc_in = jnp.concatenate([_bf(sem_row), cond_row], axis=1)
    r1 = _bf(relu(_dotT(rc_in, rc_w1_ref[...]) + rc_b1r_ref[...]))
    r2 = _bf(relu(_dotT(r1, rc_w2_ref[...]) + rc_b2r_ref[...]))
    raw = 1.0 / (1.0 + jnp.exp(-(_dotT(r2, rc_w3_ref[...])[0, 0] + rc_b3)))
    dyn = MIN_CR + (MAX_CR - MIN_CR) * raw
    cr = jnp.clip((1.0 - ALPHA) * base_cr + ALPHA * dyn, MIN_CR, MAX_CR)
    cr_safe = jnp.clip(cr, 0.001, 1.0)
    kf = jnp.clip(jnp.round(cr_safe * float(C)), 1.0, float(C))

    w_col = lax.dot_general(eye_ref[...], w_row, (((1,), (1,)), ((), ())),
                            precision=lax.Precision.HIGHEST,
                            preferred_element_type=f32)
    beats = ((w_row > w_col).astype(f32)
             + (w_row == w_col).astype(f32) * lt_ref[...])
    rank_row = lax.dot_general(ones_ref[...], beats,
                               (((1,), (1,)), ((), ())),
                               preferred_element_type=f32)
    mask_row = (rank_row < kf).astype(f32)

    out_ref[0] = xb * mask_row
    cr_ref[...] = jnp.full((1, 8, 128), cr, f32)


_IDX = np.arange(C)
_EYE = (_IDX[:, None] == _IDX[None, :]).astype(np.float32)
_LT = (_IDX[None, :] < _IDX[:, None]).astype(np.float32)
_ONES_ROW = np.ones((1, C), np.float32)


def kernel(x, channel_embed, snr_w1, snr_b1, snr_w2, snr_b2, sem_proj_w,
           cond_proj_w, out_proj_w, rc_w1, rc_b1, rc_w2, rc_b2, rc_w3, rc_b3,
           base_cr, snr_db):
    f32 = jnp.float32
    xt = jnp.transpose(x, (0, 2, 3, 1)).reshape(B, HW, C)
    scal = jnp.stack([jnp.asarray(base_cr, f32), jnp.asarray(snr_db, f32),
                      rc_b3.astype(f32)[0]]).reshape(1, 3)
    row = lambda v: v.reshape(1, -1)

    full = lambda s: pl.BlockSpec(s, lambda i: (0,) * len(s))
    out3, cr_buf = pl.pallas_call(
        _body,
        grid=(B,),
        in_specs=[
            pl.BlockSpec(memory_space=pltpu.SMEM),
            pl.BlockSpec((1, HW, C), lambda i: (i, 0, 0)),
            full(channel_embed.shape), full((1, 16)), full((1, 16)),
            full(snr_w2.shape), full((1, 16)),
            full(sem_proj_w.shape), full(cond_proj_w.shape),
            full(out_proj_w.shape),
            full(rc_w1.shape), full((1, rc_b1.shape[0])), full(rc_w2.shape),
            full((1, rc_b2.shape[0])), full(rc_w3.shape),
            full((C, C)), full((C, C)), full((1, C)),
        ],
        out_specs=[
            pl.BlockSpec((1, HW, C), lambda i: (i, 0, 0)),
            pl.BlockSpec((1, 8, 128), lambda i: (i, 0, 0)),
        ],
        out_shape=[
            jax.ShapeDtypeStruct((B, HW, C), f32),
            jax.ShapeDtypeStruct((B, 8, 128), f32),
        ],
        compiler_params=pltpu.CompilerParams(
            dimension_semantics=("arbitrary",)),
    )(scal, xt, channel_embed, row(snr_w1[:, 0]), row(snr_b1), snr_w2,
      row(snr_b2), sem_proj_w, cond_proj_w, out_proj_w,
      rc_w1, row(rc_b1), rc_w2, row(rc_b2), rc_w3,
      _EYE, _LT, _ONES_ROW)

    out = jnp.transpose(out3.reshape(B, H, W, C), (0, 3, 1, 2))
    return out, cr_buf[:, 0, 0]


# speedup vs baseline: 1.0274x; 1.0009x over previous
import jax
import jax.numpy as jnp
import numpy as np
from jax import lax
from jax.experimental import pallas as pl
from jax.experimental.pallas import tpu as pltpu

B, C, H, W = 32, 768, 32, 32
HW = H * W
CHANNEL_IDX = 0
MIN_CR, MAX_CR, ALPHA = 0.3, 1.0, 0.7


def _dotT(a, b):
    return lax.dot_general(a, b, (((1,), (1,)), ((), ())),
                           preferred_element_type=jnp.float32)


def _bf(v):
    return v.astype(jnp.bfloat16).astype(jnp.float32)


def _body(scal_ref, x_ref, cemb_ref, snr_w1r_ref, snr_b1r_ref, snr_w2_ref,
          snr_b2r_ref, sem_w_ref, cond_w_ref, out_w_ref, rc_w1_ref,
          rc_b1r_ref, rc_w2_ref, rc_b2r_ref, rc_w3_ref, eye_ref, lt_ref,
          ones_ref, out_ref, cr_ref):
    f32 = jnp.float32
    base_cr = scal_ref[0, 0]
    snr_db = scal_ref[0, 1]
    rc_b3 = scal_ref[0, 2]
    relu = lambda v: jnp.maximum(v, 0.0)

    xb = x_ref[0]
    sem_row = jnp.sum(xb, axis=0, keepdims=True) * f32(1.0 / HW)

    ch_row = _bf(cemb_ref[CHANNEL_IDX:CHANNEL_IDX + 1, :])
    snr_norm = snr_db / 28.0
    h1 = _bf(relu(snr_w1r_ref[...] * snr_norm + snr_b1r_ref[...]))
    snr_row = _bf(relu(_dotT(h1, snr_w2_ref[...]) + snr_b2r_ref[...]))
    cond_row = jnp.concatenate([ch_row, snr_row], axis=1)

    fused = _bf(relu(_dotT(sem_row, sem_w_ref[...])
                     + _dotT(cond_row, cond_w_ref[...])))
    w_row = 1.0 / (1.0 + jnp.exp(-_dotT(fused, out_w_ref[...])))

    rc_in = jnp.concatenate([_bf(sem_row), cond_row], axis=1)
    r1 = _bf(relu(_dotT(rc_in, rc_w1_ref[...]) + rc_b1r_ref[...]))
    r2 = _bf(relu(_dotT(r1, rc_w2_ref[...]) + rc_b2r_ref[...]))
    raw = 1.0 / (1.0 + jnp.exp(-(_dotT(r2, rc_w3_ref[...])[0, 0] + rc_b3)))
    dyn = MIN_CR + (MAX_CR - MIN_CR) * raw
    cr = jnp.clip((1.0 - ALPHA) * base_cr + ALPHA * dyn, MIN_CR, MAX_CR)
    cr_safe = jnp.clip(cr, 0.001, 1.0)
    kf = jnp.clip(jnp.round(cr_safe * float(C)), 1.0, float(C))

    w_col = lax.dot_general(eye_ref[...], w_row, (((1,), (1,)), ((), ())),
                            precision=lax.Precision.HIGHEST,
                            preferred_element_type=f32)
    beats = ((w_row > w_col).astype(f32)
             + (w_row == w_col).astype(f32) * lt_ref[...])
    rank_row = lax.dot_general(ones_ref[...], beats,
                               (((1,), (1,)), ((), ())),
                               preferred_element_type=f32)
    mask_row = (rank_row < kf).astype(f32)

    out_ref[0] = xb * mask_row
    cr_ref[...] = jnp.full((1, 8, 128), cr, f32)


_IDX = np.arange(C)
_EYE = (_IDX[:, None] == _IDX[None, :]).astype(np.float32)
_LT = (_IDX[None, :] < _IDX[:, None]).astype(np.float32)
_ONES_ROW = np.ones((1, C), np.float32)


def kernel(x, channel_embed, snr_w1, snr_b1, snr_w2, snr_b2, sem_proj_w,
           cond_proj_w, out_proj_w, rc_w1, rc_b1, rc_w2, rc_b2, rc_w3, rc_b3,
           base_cr, snr_db):
    f32 = jnp.float32
    xt = jnp.transpose(x, (0, 2, 3, 1)).reshape(B, HW, C)
    scal = jnp.stack([jnp.asarray(base_cr, f32), jnp.asarray(snr_db, f32),
                      rc_b3.astype(f32)[0]]).reshape(1, 3)
    row = lambda v: v.reshape(1, -1)

    full = lambda s: pl.BlockSpec(s, lambda i: (0,) * len(s))
    out3, cr_buf = pl.pallas_call(
        _body,
        grid=(B,),
        in_specs=[
            pl.BlockSpec(memory_space=pltpu.SMEM),
            pl.BlockSpec((1, HW, C), lambda i: (i, 0, 0)),
            full(channel_embed.shape), full((1, 16)), full((1, 16)),
            full(snr_w2.shape), full((1, 16)),
            full(sem_proj_w.shape), full(cond_proj_w.shape),
            full(out_proj_w.shape),
            full(rc_w1.shape), full((1, rc_b1.shape[0])), full(rc_w2.shape),
            full((1, rc_b2.shape[0])), full(rc_w3.shape),
            full((C, C)), full((C, C)), full((1, C)),
        ],
        out_specs=[
            pl.BlockSpec((1, HW, C), lambda i: (i, 0, 0)),
            pl.BlockSpec((1, 8, 128), lambda i: (i, 0, 0)),
        ],
        out_shape=[
            jax.ShapeDtypeStruct((B, HW, C), f32),
            jax.ShapeDtypeStruct((B, 8, 128), f32),
        ],
        compiler_params=pltpu.CompilerParams(
            dimension_semantics=("parallel",)),
    )(scal, xt, channel_embed, row(snr_w1[:, 0]), row(snr_b1), snr_w2,
      row(snr_b2), sem_proj_w, cond_proj_w, out_proj_w,
      rc_w1, row(rc_b1), rc_w2, row(rc_b2), rc_w3,
      _EYE, _LT, _ONES_ROW)

    out = jnp.transpose(out3.reshape(B, H, W, C), (0, 3, 1, 2))
    return out, cr_buf[:, 0, 0]
